# baseline (speedup 1.0000x reference)
import functools

import jax
import jax.numpy as jnp
from jax import lax
from jax.experimental import pallas as pl
from jax.experimental.pallas import tpu as pltpu

_BLOCK = 16384
_K = 8
_M = 6


def _tail_body(main_hbm, tailval_ref, out_ref):
    out_ref[...] = tailval_ref[...]


def _body(in_hbm, x_ref, tail_ref, out_hbm, tailval_ref, cache_ref, m_ref,
          s_ref, lse_ref, tmp_ref, insem, outsem, *, n_cols, block, nb):
    p = pl.program_id(0)
    j = pl.program_id(1)

    def read_copy(b, slot):
        return pltpu.make_async_copy(
            in_hbm.at[:, pl.ds(b * block, block)],
            cache_ref.at[:, pl.ds(b * block, block)],
            insem.at[slot])

    @pl.when(p == 0)
    def _reduce_phase():
        @pl.when(j == 0)
        def _prime():
            for b in range(_K):
                read_copy(b, b).start()

        @pl.when((j > 0) & (j + _K - 1 < nb - 1))
        def _next():
            b = j + _K - 1
            read_copy(b, lax.rem(b, _K)).start()

        @pl.when(j < nb - 1)
        def _wait_full():
            read_copy(j, lax.rem(j, _K)).wait()

        def update(mblk):
            bm = jnp.max(mblk, axis=1, keepdims=True)
            neg_inf = jnp.full(m_ref.shape, -jnp.inf, m_ref.dtype)
            m_old = jnp.where(j == 0, neg_inf, m_ref[...])
            s_old = jnp.where(j == 0, jnp.zeros_like(s_ref), s_ref[...])
            m_new = jnp.maximum(m_old, bm)
            s_new = (s_old * jnp.exp(m_old - m_new)
                     + jnp.sum(jnp.exp(mblk - m_new), axis=1, keepdims=True))
            m_ref[...] = m_new
            s_ref[...] = s_new
            return m_new, s_new

        @pl.when(j < nb - 1)
        def _full():
            update(cache_ref[:, pl.ds(j * block, block)])

        @pl.when(j == nb - 1)
        def _ragged():
            blk = tail_ref[...]
            cache_ref[:, pl.ds(j * block, block)] = blk
            valid = n_cols - j * block
            col = lax.broadcasted_iota(jnp.int32, blk.shape, 1)
            m_new, s_new = update(jnp.where(col < valid, blk, -jnp.inf))
            lse_ref[...] = m_new + jnp.log(s_new)

    @pl.when(p == 1)
    def _emit_phase():
        slot = lax.rem(j, _M)

        def write_copy(b, s):
            return pltpu.make_async_copy(
                tmp_ref.at[s, :, :],
                out_hbm.at[:, pl.ds(b * block, block)],
                outsem.at[s])

        @pl.when((j >= _M) & (j < nb - 1))
        def _reuse():
            pltpu.make_async_copy(
                tmp_ref.at[slot, :, :],
                out_hbm.at[:, pl.ds((j - _M) * block, block)],
                outsem.at[slot]).wait()

        xv = x_ref[...]
        k_iota = lax.broadcasted_iota(jnp.int32, (xv.shape[0],) * 2, 1)
        onehot = (xv == k_iota).astype(jnp.float32)
        blk = cache_ref[:, pl.ds(j * block, block)]
        gathered = lax.dot_general(
            onehot, blk, (((1,), (0,)), ((), ())),
            preferred_element_type=jnp.float32)
        val = gathered - lse_ref[...]

        @pl.when(j < nb - 1)
        def _issue():
            tmp_ref[slot, :, :] = val
            pltpu.make_async_copy(
                tmp_ref.at[slot, :, :],
                out_hbm.at[:, pl.ds(j * block, block)],
                outsem.at[slot]).start()

        @pl.when(j == nb - 1)
        def _last_and_drain():
            tailval_ref[...] = val
            for s in range(_M):
                b = nb - 2 - ((nb - 2 - s) % _M)
                write_copy(b, s).wait()


def kernel(logits, x):
    r, n = logits.shape
    block = _BLOCK
    nb = pl.cdiv(n, block)
    x2 = x.reshape(r, 1).astype(jnp.int32)
    out = pl.pallas_call(
        functools.partial(_body, n_cols=n, block=block, nb=nb),
        grid=(2, nb),
        in_specs=[
            pl.BlockSpec(memory_space=pl.ANY),
            pl.BlockSpec((r, 1), lambda p, j: (0, 0)),
            pl.BlockSpec((r, block), lambda p, j: (0, nb - 1)),
        ],
        out_specs=[
            pl.BlockSpec(memory_space=pl.ANY),
            pl.BlockSpec((r, block), lambda p, j: (0, 0)),
        ],
        out_shape=[jax.ShapeDtypeStruct((r, n), jnp.float32),
                   jax.ShapeDtypeStruct((r, block), jnp.float32)],
        scratch_shapes=[
            pltpu.VMEM((r, nb * block), jnp.float32),
            pltpu.VMEM((r, 1), jnp.float32),
            pltpu.VMEM((r, 1), jnp.float32),
            pltpu.VMEM((r, 1), jnp.float32),
            pltpu.VMEM((_M, r, block), jnp.float32),
            pltpu.SemaphoreType.DMA((_K,)),
            pltpu.SemaphoreType.DMA((_M,)),
        ],
        compiler_params=pltpu.CompilerParams(
            dimension_semantics=("arbitrary", "arbitrary"),
            vmem_limit_bytes=100 * 1024 * 1024,
        ),
    )(logits, x2, logits)
    out_main, tailvals = out
    out_final = pl.pallas_call(
        _tail_body,
        grid=(1,),
        in_specs=[pl.BlockSpec(memory_space=pl.ANY),
                  pl.BlockSpec((r, block), lambda i: (0, 0))],
        out_specs=pl.BlockSpec((r, block), lambda i, _nb=nb: (0, _nb - 1)),
        out_shape=jax.ShapeDtypeStruct((r, n), jnp.float32),
        input_output_aliases={0: 0},
    )(out_main, tailvals)
    return out_final


# speedup vs baseline: 1.0106x; 1.0106x over previous
import functools

import jax
import jax.numpy as jnp
from jax import lax
from jax.experimental import pallas as pl
from jax.experimental.pallas import tpu as pltpu

_BLOCK = 16384
_K = 6
_M = 4


def _tail_body(main_hbm, tailval_ref, out_ref):
    out_ref[...] = tailval_ref[...]


def _body(in_hbm, x_ref, tail_ref, out_hbm, tailval_ref, cache_ref, m_ref,
          s_ref, lse_ref, tmp_ref, insem, outsem, *, n_cols, block, nb):
    p = pl.program_id(0)
    j = pl.program_id(1)

    def read_copy(b, slot):
        return pltpu.make_async_copy(
            in_hbm.at[:, pl.ds(b * block, block)],
            cache_ref.at[:, pl.ds(b * block, block)],
            insem.at[slot])

    @pl.when(p == 0)
    def _reduce_phase():
        @pl.when(j == 0)
        def _prime():
            for b in range(_K):
                read_copy(b, b).start()

        @pl.when((j > 0) & (j + _K - 1 < nb - 1))
        def _next():
            b = j + _K - 1
            read_copy(b, lax.rem(b, _K)).start()

        @pl.when(j < nb - 1)
        def _wait_full():
            read_copy(j, lax.rem(j, _K)).wait()

        def update(mblk):
            bm = jnp.max(mblk, axis=1, keepdims=True)
            neg_inf = jnp.full(m_ref.shape, -jnp.inf, m_ref.dtype)
            m_old = jnp.where(j == 0, neg_inf, m_ref[...])
            s_old = jnp.where(j == 0, jnp.zeros_like(s_ref), s_ref[...])
            m_new = jnp.maximum(m_old, bm)
            s_new = (s_old * jnp.exp(m_old - m_new)
                     + jnp.sum(jnp.exp(mblk - m_new), axis=1, keepdims=True))
            m_ref[...] = m_new
            s_ref[...] = s_new
            return m_new, s_new

        @pl.when(j < nb - 1)
        def _full():
            update(cache_ref[:, pl.ds(j * block, block)])

        @pl.when(j == nb - 1)
        def _ragged():
            blk = tail_ref[...]
            cache_ref[:, pl.ds(j * block, block)] = blk
            valid = n_cols - j * block
            col = lax.broadcasted_iota(jnp.int32, blk.shape, 1)
            m_new, s_new = update(jnp.where(col < valid, blk, -jnp.inf))
            lse_ref[...] = m_new + jnp.log(s_new)

    @pl.when(p == 1)
    def _emit_phase():
        slot = lax.rem(j, _M)

        def write_copy(b, s):
            return pltpu.make_async_copy(
                tmp_ref.at[s, :, :],
                out_hbm.at[:, pl.ds(b * block, block)],
                outsem.at[s])

        @pl.when((j >= _M) & (j < nb - 1))
        def _reuse():
            pltpu.make_async_copy(
                tmp_ref.at[slot, :, :],
                out_hbm.at[:, pl.ds((j - _M) * block, block)],
                outsem.at[slot]).wait()

        xv = x_ref[...]
        k_iota = lax.broadcasted_iota(jnp.int32, (xv.shape[0],) * 2, 1)
        onehot = (xv == k_iota).astype(jnp.float32)
        blk = cache_ref[:, pl.ds(j * block, block)]
        gathered = lax.dot_general(
            onehot, blk, (((1,), (0,)), ((), ())),
            preferred_element_type=jnp.float32)
        val = gathered - lse_ref[...]

        @pl.when(j < nb - 1)
        def _issue():
            tmp_ref[slot, :, :] = val
            pltpu.make_async_copy(
                tmp_ref.at[slot, :, :],
                out_hbm.at[:, pl.ds(j * block, block)],
                outsem.at[slot]).start()

        @pl.when(j == nb - 1)
        def _last_and_drain():
            tailval_ref[...] = val
            for s in range(_M):
                b = nb - 2 - ((nb - 2 - s) % _M)
                write_copy(b, s).wait()


def kernel(logits, x):
    r, n = logits.shape
    block = _BLOCK
    nb = pl.cdiv(n, block)
    x2 = x.reshape(r, 1).astype(jnp.int32)
    out = pl.pallas_call(
        functools.partial(_body, n_cols=n, block=block, nb=nb),
        grid=(2, nb),
        in_specs=[
            pl.BlockSpec(memory_space=pl.ANY),
            pl.BlockSpec((r, 1), lambda p, j: (0, 0)),
            pl.BlockSpec((r, block), lambda p, j: (0, nb - 1)),
        ],
        out_specs=[
            pl.BlockSpec(memory_space=pl.ANY),
            pl.BlockSpec((r, block), lambda p, j: (0, 0)),
        ],
        out_shape=[jax.ShapeDtypeStruct((r, n), jnp.float32),
                   jax.ShapeDtypeStruct((r, block), jnp.float32)],
        scratch_shapes=[
            pltpu.VMEM((r, nb * block), jnp.float32),
            pltpu.VMEM((r, 1), jnp.float32),
            pltpu.VMEM((r, 1), jnp.float32),
            pltpu.VMEM((r, 1), jnp.float32),
            pltpu.VMEM((_M, r, block), jnp.float32),
            pltpu.SemaphoreType.DMA((_K,)),
            pltpu.SemaphoreType.DMA((_M,)),
        ],
        compiler_params=pltpu.CompilerParams(
            dimension_semantics=("arbitrary", "arbitrary"),
            vmem_limit_bytes=100 * 1024 * 1024,
        ),
    )(logits, x2, logits)
    out_main, tailvals = out
    out_final = pl.pallas_call(
        _tail_body,
        grid=(1,),
        in_specs=[pl.BlockSpec(memory_space=pl.ANY),
                  pl.BlockSpec((r, block), lambda i: (0, 0))],
        out_specs=pl.BlockSpec((r, block), lambda i, _nb=nb: (0, _nb - 1)),
        out_shape=jax.ShapeDtypeStruct((r, n), jnp.float32),
        input_output_aliases={0: 0},
    )(out_main, tailvals)
    return out_final
